# baseline (speedup 1.0000x reference)
import jax
import jax.numpy as jnp
from jax import lax
from jax.experimental import pallas as pl
from jax.experimental.pallas import tpu as pltpu

_B = 4
_T = 16
_FRAME = 224 * 224 * 3
_SEQ = _T * _FRAME
_ROWS = _SEQ // 128
_BLK = 2688
_NBLK = _ROWS // _BLK


def _body(start_ref, end_ref, cm_ref, tm_ref, tp_ref):
    b = pl.program_id(0)
    c = pl.program_id(1)
    base = c * (_BLK * 128)
    shp = (1, _BLK, 128)
    idx = (base
           + lax.broadcasted_iota(jnp.int32, shp, 1) * 128
           + lax.broadcasted_iota(jnp.int32, shp, 2))
    s = start_ref[b]
    e = end_ref[b]
    tm = (idx >= s) & (idx < e)
    tm_ref[...] = tm
    cm_ref[...] = ~tm
    tp_ref[...] = jnp.where(idx < (e - s), s + idx, _SEQ)


def kernel(batch_size, num_frames, frame_size, scales, rand_start):
    num_mask = jnp.clip((scales * _T).astype(jnp.int32), 1, _T - 2)
    max_start = jnp.clip(_T - num_mask - 1, 1, None)
    start_frames = (rand_start * max_start.astype(jnp.float32) + 1.0).astype(jnp.int32)
    start_pos = start_frames * _FRAME
    end_pos = jnp.minimum((start_frames + num_mask) * _FRAME, _SEQ)

    cm, tm, tp = pl.pallas_call(
        _body,
        grid=(_B, _NBLK),
        in_specs=[
            pl.BlockSpec(memory_space=pltpu.SMEM),
            pl.BlockSpec(memory_space=pltpu.SMEM),
        ],
        out_specs=[
            pl.BlockSpec((1, _BLK, 128), lambda b, c: (b, c, 0)),
            pl.BlockSpec((1, _BLK, 128), lambda b, c: (b, c, 0)),
            pl.BlockSpec((1, _BLK, 128), lambda b, c: (b, c, 0)),
        ],
        out_shape=[
            jax.ShapeDtypeStruct((_B, _ROWS, 128), jnp.bool_),
            jax.ShapeDtypeStruct((_B, _ROWS, 128), jnp.bool_),
            jax.ShapeDtypeStruct((_B, _ROWS, 128), jnp.int32),
        ],
    )(start_pos, end_pos)
    return (cm.reshape(_B, _SEQ), tm.reshape(_B, _SEQ), tp.reshape(_B, _SEQ))


# speedup vs baseline: 3.3696x; 3.3696x over previous
import jax
import jax.numpy as jnp
from jax import lax
from jax.experimental import pallas as pl
from jax.experimental.pallas import tpu as pltpu

_B = 4
_T = 16
_FRAME = 224 * 224 * 3
_SEQ = _T * _FRAME
_CHUNK = 114688
_NCHUNK = _SEQ // _CHUNK


def _body(start_ref, end_ref, cm_ref, tm_ref, tp_ref):
    c = pl.program_id(0)
    base = c * _CHUNK
    idx = base + lax.broadcasted_iota(jnp.int32, (_B, _CHUNK), 1)
    row = lax.broadcasted_iota(jnp.int32, (_B, _CHUNK), 0)

    def per_row(vals_ref):
        v0, v1, v2, v3 = vals_ref[0], vals_ref[1], vals_ref[2], vals_ref[3]
        return jnp.where(row == 0, v0,
               jnp.where(row == 1, v1,
               jnp.where(row == 2, v2, v3)))

    s = per_row(start_ref)
    e = per_row(end_ref)
    tm = (idx >= s) & (idx < e)
    tm_ref[...] = tm
    cm_ref[...] = ~tm
    tp_ref[...] = jnp.where(idx < (e - s), s + idx, _SEQ)


def kernel(batch_size, num_frames, frame_size, scales, rand_start):
    num_mask = jnp.clip((scales * _T).astype(jnp.int32), 1, _T - 2)
    max_start = jnp.clip(_T - num_mask - 1, 1, None)
    start_frames = (rand_start * max_start.astype(jnp.float32) + 1.0).astype(jnp.int32)
    start_pos = start_frames * _FRAME
    end_pos = jnp.minimum((start_frames + num_mask) * _FRAME, _SEQ)

    cm, tm, tp = pl.pallas_call(
        _body,
        grid=(_NCHUNK,),
        in_specs=[
            pl.BlockSpec(memory_space=pltpu.SMEM),
            pl.BlockSpec(memory_space=pltpu.SMEM),
        ],
        out_specs=[
            pl.BlockSpec((_B, _CHUNK), lambda c: (0, c)),
            pl.BlockSpec((_B, _CHUNK), lambda c: (0, c)),
            pl.BlockSpec((_B, _CHUNK), lambda c: (0, c)),
        ],
        out_shape=[
            jax.ShapeDtypeStruct((_B, _SEQ), jnp.bool_),
            jax.ShapeDtypeStruct((_B, _SEQ), jnp.bool_),
            jax.ShapeDtypeStruct((_B, _SEQ), jnp.int32),
        ],
    )(start_pos, end_pos)
    return (cm, tm, tp)
